# Initial kernel scaffold; baseline (speedup 1.0000x reference)
import jax
import jax.numpy as jnp
from jax.experimental import pallas as pl


def kernel(x, edge_list, dummy, batch, W1, b1, p_vec, W3, b3, W2, b2):
    raise NotImplementedError("write your pallas kernel here")


# speedup vs baseline: 32.5465x; 32.5465x over previous
import dataclasses
import functools

import jax
import jax.numpy as jnp
from jax import lax
from jax.experimental import pallas as pl
from jax.experimental.pallas import tpu as pltpu
from jax.experimental.pallas import tpu_sc as plsc

N = 10000
E = 320000
D = 128
HID = 256
NG = 128
TOPK = 5000

NP = 10240
GR = NP
RS = NP + 128
NC, NS = 2, 16
NW = NC * NS
CHUNK = 10240
NB = CHUNK // 128
BB = 128
EP = NW * CHUNK

_mesh = plsc.VectorSubcoreMesh(core_axis_name="c", subcore_axis_name="s")

_cp_no_layout = pltpu.CompilerParams()
if "needs_layout_passes" in pltpu.CompilerParams.__dataclass_fields__:
    _cp_no_layout = dataclasses.replace(_cp_no_layout,
                                        needs_layout_passes=False)


def _zero16(dtype):
    return jnp.zeros((16,), dtype)


def _zero_1d(ref):
    z = _zero16(ref.dtype)

    @pl.loop(0, ref.shape[0] // 16)
    def _(i):
        ref[pl.ds(i * 16, 16)] = z


@functools.partial(
    pl.kernel,
    mesh=_mesh,
    compiler_params=_cp_no_layout,
    out_type=jax.ShapeDtypeStruct((NW, RS), jnp.float32),
    scratch_types=[
        pltpu.VMEM((RS,), jnp.float32),
        pltpu.VMEM((BB,), jnp.int32),
    ],
)
def _k1_deg(dst_hbm, deg_hbm, deg_v, dbuf):
    c = lax.axis_index("c")
    s = lax.axis_index("s")
    wid = s * NC + c
    _zero_1d(deg_v)
    ones16 = jnp.ones((16,), jnp.float32)
    base = wid * CHUNK

    @pl.loop(0, NB)
    def _(b):
        pltpu.sync_copy(dst_hbm.at[pl.ds(base + b * BB, BB)], dbuf)
        for j in range(BB // 16):
            d16 = dbuf[pl.ds(j * 16, 16)]
            plsc.addupdate_scatter(deg_v, [d16], ones16)

    pltpu.sync_copy(deg_v, deg_hbm.at[wid])


@functools.partial(
    pl.kernel,
    mesh=_mesh,
    out_type=jax.ShapeDtypeStruct((NC, NP, D), jnp.float32),
    scratch_types=[
        pltpu.VMEM((BB,), jnp.int32),
        pltpu.VMEM((BB,), jnp.int32),
        pltpu.VMEM((BB,), jnp.int32),
        pltpu.VMEM((BB,), jnp.int32),
        pltpu.VMEM((BB, D), jnp.float32),
        pltpu.VMEM((BB, D), jnp.float32),
        pltpu.VMEM((16, D), jnp.float32),
        pltpu.VMEM_SHARED((RS, D), jnp.float32),
        pltpu.SemaphoreType.DMA,
        pltpu.SemaphoreType.DMA,
    ],
)
def _k3_agg1(y_hbm, src_hbm, dst_hbm, zsl_hbm, agg_hbm, sidx0, sidx1,
             didx0, didx1, rows0, rows1, z_v, agg_sh, sem0, sem1):
    c = lax.axis_index("c")
    s = lax.axis_index("s")
    wid = s * NC + c
    pltpu.sync_copy(zsl_hbm, z_v)

    @pl.loop(s, RS // 16, step=NS)
    def _(j):
        pltpu.sync_copy(z_v, agg_sh.at[pl.ds(j * 16, 16)])

    plsc.subcore_barrier()
    base = wid * CHUNK
    sidx = (sidx0, sidx1)
    didx = (didx0, didx1)
    rows = (rows0, rows1)
    sems = (sem0, sem1)

    pltpu.sync_copy(src_hbm.at[pl.ds(base, BB)], sidx[0])
    pltpu.sync_copy(dst_hbm.at[pl.ds(base, BB)], didx[0])
    pltpu.async_copy(y_hbm.at[sidx[0]], rows[0], sems[0])

    @pl.loop(0, NB, step=2)
    def _(b):
        for t in range(2):
            cur, nxt = t, 1 - t
            bb = b + t

            @pl.when(bb + 1 < NB)
            def _():
                off = base + (bb + 1) * BB
                pltpu.sync_copy(src_hbm.at[pl.ds(off, BB)], sidx[nxt])
                pltpu.sync_copy(dst_hbm.at[pl.ds(off, BB)], didx[nxt])
                pltpu.async_copy(y_hbm.at[sidx[nxt]], rows[nxt], sems[nxt])

            pltpu.make_async_copy(y_hbm.at[sidx[cur]], rows[cur],
                                  sems[cur]).wait()
            pltpu.sync_copy(rows[cur], agg_sh.at[didx[cur]], add=True)

    plsc.subcore_barrier()
    rows_per = NP // NS
    pltpu.sync_copy(agg_sh.at[pl.ds(s * rows_per, rows_per)],
                    agg_hbm.at[c].at[pl.ds(s * rows_per, rows_per)])


@functools.partial(
    pl.kernel,
    mesh=_mesh,
    compiler_params=_cp_no_layout,
    out_type=(
        jax.ShapeDtypeStruct((NW, CHUNK), jnp.int32),
        jax.ShapeDtypeStruct((NW, CHUNK), jnp.int32),
        jax.ShapeDtypeStruct((NW, 16), jnp.int32),
        jax.ShapeDtypeStruct((NW, RS), jnp.float32),
    ),
    scratch_types=[
        pltpu.VMEM((NP + 16,), jnp.int32),
        pltpu.VMEM((BB,), jnp.int32),
        pltpu.VMEM((BB,), jnp.int32),
        pltpu.VMEM((CHUNK,), jnp.int32),
        pltpu.VMEM((CHUNK,), jnp.int32),
        pltpu.VMEM((RS,), jnp.float32),
        pltpu.VMEM((16,), jnp.int32),
    ],
)
def _k5_compact(src_hbm, dst_hbm, kept_hbm, csrc_hbm, cdst_hbm, cnt_hbm,
                deg_hbm, kept_v, sbuf, dbuf, csrc_v, cdst_v, deg_v, cnt_v):
    c = lax.axis_index("c")
    s = lax.axis_index("s")
    wid = s * NC + c
    lane = lax.iota(jnp.int32, 16)
    _zero_1d(deg_v)
    pltpu.sync_copy(kept_hbm, kept_v.at[pl.ds(0, NP)])
    kept_v[pl.ds(NP, 16)] = _zero16(jnp.int32)
    zi = _zero16(jnp.int32)
    gi = jnp.full((16,), GR, jnp.int32)
    ones16 = jnp.ones((16,), jnp.float32)

    @pl.loop(0, CHUNK // 16)
    def _(i):
        csrc_v[pl.ds(i * 16, 16)] = zi
        cdst_v[pl.ds(i * 16, 16)] = gi

    base = wid * CHUNK

    def batch_body(b, cnt):
        pltpu.sync_copy(src_hbm.at[pl.ds(base + b * BB, BB)], sbuf)
        pltpu.sync_copy(dst_hbm.at[pl.ds(base + b * BB, BB)], dbuf)
        for j in range(BB // 16):
            s16 = sbuf[pl.ds(j * 16, 16)]
            d16 = dbuf[pl.ds(j * 16, 16)]
            ks = plsc.load_gather(kept_v, [s16])
            kd = plsc.load_gather(kept_v, [d16])
            vi = ks * kd
            valid = vi == 1
            pos = plsc.cumsum(vi)
            idx16 = cnt + pos - 1
            plsc.store_scatter(csrc_v, [idx16], s16, mask=valid)
            plsc.store_scatter(cdst_v, [idx16], d16, mask=valid)
            plsc.addupdate_scatter(deg_v, [jnp.where(valid, d16, GR)],
                                   ones16)
            cnt = cnt + jnp.sum(vi)
        return cnt

    total = lax.fori_loop(0, NB, batch_body, jnp.int32(0))
    cnt_v[...] = jnp.where(lane == 0, total, 0).astype(jnp.int32)
    pltpu.sync_copy(csrc_v, csrc_hbm.at[wid])
    pltpu.sync_copy(cdst_v, cdst_hbm.at[wid])
    pltpu.sync_copy(cnt_v, cnt_hbm.at[wid])
    pltpu.sync_copy(deg_v, deg_hbm.at[wid])


@functools.partial(
    pl.kernel,
    mesh=_mesh,
    out_type=jax.ShapeDtypeStruct((NC, NP, D), jnp.float32),
    scratch_types=[
        pltpu.VMEM((BB,), jnp.int32),
        pltpu.VMEM((BB,), jnp.int32),
        pltpu.VMEM((BB,), jnp.int32),
        pltpu.VMEM((BB,), jnp.int32),
        pltpu.VMEM((BB, D), jnp.float32),
        pltpu.VMEM((BB, D), jnp.float32),
        pltpu.VMEM((16, D), jnp.float32),
        pltpu.VMEM((16,), jnp.int32),
        pltpu.VMEM_SHARED((RS, D), jnp.float32),
        pltpu.SemaphoreType.DMA,
        pltpu.SemaphoreType.DMA,
    ],
)
def _k7_agg2(y2_hbm, csrc_hbm, cdst_hbm, cnt_hbm, zsl_hbm, agg_hbm,
             sidx0, sidx1, didx0, didx1, rows0, rows1, z_v, cnt_v,
             agg_sh, sem0, sem1):
    c = lax.axis_index("c")
    s = lax.axis_index("s")
    pltpu.sync_copy(zsl_hbm, z_v)

    @pl.loop(s, RS // 16, step=NS)
    def _(j):
        pltpu.sync_copy(z_v, agg_sh.at[pl.ds(j * 16, 16)])

    plsc.subcore_barrier()
    sidx = (sidx0, sidx1)
    didx = (didx0, didx1)
    rows = (rows0, rows1)
    sems = (sem0, sem1)

    for t in range(2):
        w = s * 2 + t
        pltpu.sync_copy(cnt_hbm.at[w], cnt_v)
        cw = cnt_v[...][0]
        npair = (cw + 255) // 256

        @pl.when(npair > 0)
        def _():
            pltpu.sync_copy(csrc_hbm.at[w, pl.ds(0, BB)], sidx[0])
            pltpu.sync_copy(cdst_hbm.at[w, pl.ds(0, BB)], didx[0])
            pltpu.async_copy(y2_hbm.at[c].at[sidx[0]], rows[0], sems[0])

            @pl.loop(0, npair)
            def _(p):
                for u in range(2):
                    bb = p * 2 + u
                    cur, nxt = u, 1 - u

                    @pl.when(bb + 1 < npair * 2)
                    def _():
                        off = (bb + 1) * BB
                        pltpu.sync_copy(csrc_hbm.at[w, pl.ds(off, BB)],
                                        sidx[nxt])
                        pltpu.sync_copy(cdst_hbm.at[w, pl.ds(off, BB)],
                                        didx[nxt])
                        pltpu.async_copy(y2_hbm.at[c].at[sidx[nxt]],
                                         rows[nxt], sems[nxt])

                    pltpu.make_async_copy(y2_hbm.at[c].at[sidx[cur]],
                                          rows[cur], sems[cur]).wait()
                    pltpu.sync_copy(rows[cur], agg_sh.at[didx[cur]],
                                    add=True)

    plsc.subcore_barrier()
    rows_per = NP // NS
    pltpu.sync_copy(agg_sh.at[pl.ds(s * rows_per, rows_per)],
                    agg_hbm.at[c].at[pl.ds(s * rows_per, rows_per)])


_RB = 1280
_GRID = NP // _RB


def _sum_partials(degp_blk):
    ones = jnp.ones((NW, 1), jnp.float32)
    return lax.dot_general(degp_blk, ones, (((0,), (0,)), ((), ())),
                           preferred_element_type=jnp.float32,
                           precision=lax.Precision.HIGHEST)


def _k2_body(degp_ref, x_ref, y_ref):
    dinv = lax.rsqrt(jnp.maximum(_sum_partials(degp_ref[...]) + 1.0, 1.0))
    y_ref[...] = dinv * x_ref[...]


def _k4a_body(x_ref, aggp_ref, degp_ref, w1_ref, b1_ref, p_ref, h1_ref,
              sc_ref):
    i = pl.program_id(0)
    dinv = lax.rsqrt(jnp.maximum(_sum_partials(degp_ref[...]) + 1.0, 1.0))
    u = dinv * (aggp_ref[0] + aggp_ref[1]) + (dinv * dinv) * x_ref[...]
    h1 = jnp.maximum(
        lax.dot_general(u, w1_ref[...], (((1,), (0,)), ((), ())),
                        preferred_element_type=jnp.float32,
                        precision=lax.Precision.HIGHEST)
        + b1_ref[...][None, :], 0.0)
    h1_ref[...] = h1
    p = p_ref[...]
    pn = jnp.sqrt(jnp.sum(p * p))
    sc = lax.dot_general(h1, p, (((1,), (0,)), ((), ())),
                         preferred_element_type=jnp.float32,
                         precision=lax.Precision.HIGHEST) / pn
    ridx = i * _RB + lax.broadcasted_iota(jnp.int32, (_RB, 1), 0)
    sc_ref[...] = jnp.where(ridx < N, sc, -jnp.inf)


def _k4b_body(sc_ref, kept_ref, gs_ref):
    s = sc_ref[...]
    bits = lax.bitcast_convert_type(s, jnp.int32)
    m = bits ^ jnp.where(bits < 0, jnp.int32(0x7FFFFFFF), jnp.int32(0))

    def bs(it, T):
        cand = T + (jnp.int32(1) << (jnp.int32(30) - it))
        c = jnp.sum((m >= cand).astype(jnp.int32))
        return jnp.where(c >= TOPK, cand, T)

    T = lax.fori_loop(0, 31, bs, jnp.int32(-2**31))
    cnt_gt = jnp.sum((m > T).astype(jnp.int32))
    need = (TOPK - cnt_gt).astype(jnp.float32)
    eq = (m == T)
    eqf = eq.astype(jnp.float32)
    r128 = lax.broadcasted_iota(jnp.int32, (128, 128), 0)
    c128 = lax.broadcasted_iota(jnp.int32, (128, 128), 1)
    u128 = (r128 < c128).astype(jnp.float32)
    p1 = lax.dot_general(eqf, u128, (((1,), (0,)), ((), ())),
                         preferred_element_type=jnp.float32,
                         precision=lax.Precision.HIGHEST)
    rowtot = jnp.sum(eqf, axis=1, keepdims=True)
    r80 = lax.broadcasted_iota(jnp.int32, (80, 80), 0)
    c80 = lax.broadcasted_iota(jnp.int32, (80, 80), 1)
    u80 = (r80 < c80).astype(jnp.float32)
    carry = lax.dot_general(u80, rowtot, (((1,), (0,)), ((), ())),
                            preferred_element_type=jnp.float32,
                            precision=lax.Precision.HIGHEST)
    prefix = p1 + carry
    kept = (m > T) | (eq & (prefix < need))
    keptf = kept.astype(jnp.float32)
    kept_ref[...] = keptf
    gs_ref[...] = keptf * jnp.tanh(s)


def _k4c_body(h1_ref, gs_ref, w3_ref, z_ref):
    xp = gs_ref[...] * h1_ref[...]
    z_ref[...] = lax.dot_general(xp, w3_ref[...], (((1,), (0,)), ((), ())),
                                 preferred_element_type=jnp.float32,
                                 precision=lax.Precision.HIGHEST)


def _dinv2_of(degp_blk, keptf):
    deg2 = _sum_partials(degp_blk) + keptf
    return keptf * lax.rsqrt(jnp.maximum(deg2, 1.0))


def _k6_body(z_ref, degp_ref, kept_ref, y2_ref):
    dinv2 = _dinv2_of(degp_ref[...], kept_ref[...])
    z = z_ref[...]
    y2_ref[0] = dinv2 * z[:, :D]
    y2_ref[1] = dinv2 * z[:, D:]


def _k8_body(agg2_ref, z_ref, degp_ref, kept_ref, batch_ref, b3_ref,
             w2_ref, b2_ref, out_ref, sums, cnt):
    i = pl.program_id(0)

    @pl.when(i == 0)
    def _():
        sums[...] = jnp.zeros_like(sums)
        cnt[...] = jnp.zeros_like(cnt)

    keptf = kept_ref[...]
    dinv2 = _dinv2_of(degp_ref[...], keptf)
    agg = jnp.concatenate([agg2_ref[0], agg2_ref[1]], axis=1)
    h3 = jnp.maximum(dinv2 * agg + (dinv2 * dinv2) * z_ref[...]
                     + b3_ref[...][None, :], 0.0)
    oh = (batch_ref[...] ==
          lax.broadcasted_iota(jnp.int32, (1, NG), 1)).astype(jnp.float32)
    wh3 = keptf * h3
    sums[...] += lax.dot_general(oh, wh3, (((0,), (0,)), ((), ())),
                                 preferred_element_type=jnp.float32,
                                 precision=lax.Precision.HIGHEST)
    cnt[...] += lax.dot_general(oh, keptf, (((0,), (0,)), ((), ())),
                                preferred_element_type=jnp.float32,
                                precision=lax.Precision.HIGHEST)

    @pl.when(i == _GRID - 1)
    def _():
        gm = sums[...] / jnp.maximum(cnt[...], 1.0)
        logit = lax.dot_general(gm, w2_ref[...], (((1,), (0,)), ((), ())),
                                preferred_element_type=jnp.float32,
                                precision=lax.Precision.HIGHEST)
        out_ref[...] = jax.nn.sigmoid(logit + b2_ref[...])


def _rows(block_cols):
    return pl.BlockSpec((_RB, block_cols), lambda i: (i, 0))


def _pair(block_cols):
    return pl.BlockSpec((2, _RB, block_cols), lambda i: (0, i, 0))


def _deg_spec():
    return pl.BlockSpec((NW, _RB), lambda i: (0, i))


def _full(shape):
    return pl.BlockSpec(shape, lambda i: tuple(0 for _ in shape))


def kernel(x, edge_list, dummy, batch, W1, b1, p_vec, W3, b3, W2, b2):
    f32 = jnp.float32
    src = edge_list[:, 0].astype(jnp.int32)
    dst = edge_list[:, 1].astype(jnp.int32)
    pad_e = CHUNK - E // NW
    srcp = jnp.concatenate(
        [src.reshape(NW, E // NW),
         jnp.zeros((NW, pad_e), jnp.int32)], axis=1).reshape(-1)
    dstp = jnp.concatenate(
        [dst.reshape(NW, E // NW),
         jnp.full((NW, pad_e), GR, jnp.int32)], axis=1).reshape(-1)
    xp = jnp.pad(x, ((0, NP - N), (0, 0)))
    batchp = jnp.pad(batch.astype(jnp.int32), (0, NP - N)).reshape(NP, 1)
    zsl = jnp.zeros((16, D), f32)

    deg1p = _k1_deg(dstp)

    y = pl.pallas_call(
        _k2_body,
        grid=(_GRID,),
        in_specs=[_deg_spec(), _rows(D)],
        out_specs=_rows(D),
        out_shape=jax.ShapeDtypeStruct((NP, D), f32),
    )(deg1p, xp)

    agg1p = _k3_agg1(y, srcp, dstp, zsl)

    h1, score = pl.pallas_call(
        _k4a_body,
        grid=(_GRID,),
        in_specs=[_rows(D), _pair(D), _deg_spec(), _full((D, HID)),
                  _full((HID,)), _full((HID, 1))],
        out_specs=[_rows(HID), _rows(1)],
        out_shape=[jax.ShapeDtypeStruct((NP, HID), f32),
                   jax.ShapeDtypeStruct((NP, 1), f32)],
    )(xp, agg1p, deg1p, W1, b1, p_vec.reshape(HID, 1))

    keptm, gsm = pl.pallas_call(
        _k4b_body,
        in_specs=[pl.BlockSpec((80, 128), lambda: (0, 0))],
        out_specs=[pl.BlockSpec((80, 128), lambda: (0, 0))] * 2,
        out_shape=[jax.ShapeDtypeStruct((80, 128), f32)] * 2,
    )(score.reshape(80, 128))
    keptc = keptm.reshape(NP, 1)
    kept_i = keptm.reshape(NP).astype(jnp.int32)

    z = pl.pallas_call(
        _k4c_body,
        grid=(_GRID,),
        in_specs=[_rows(HID), _rows(1), _full((HID, HID))],
        out_specs=_rows(HID),
        out_shape=jax.ShapeDtypeStruct((NP, HID), f32),
    )(h1, gsm.reshape(NP, 1), W3)

    csrc, cdst, cnts, deg2p = _k5_compact(srcp, dstp, kept_i)

    y2 = pl.pallas_call(
        _k6_body,
        grid=(_GRID,),
        in_specs=[_rows(HID), _deg_spec(), _rows(1)],
        out_specs=_pair(D),
        out_shape=jax.ShapeDtypeStruct((NC, NP, D), f32),
    )(z, deg2p, keptc)

    agg2 = _k7_agg2(y2, csrc, cdst, cnts, zsl)

    out = pl.pallas_call(
        _k8_body,
        grid=(_GRID,),
        in_specs=[_pair(D), _rows(HID), _deg_spec(), _rows(1),
                  pl.BlockSpec((_RB, 1), lambda i: (i, 0)),
                  _full((HID,)), _full((HID, 1)), _full((1, 1))],
        out_specs=_full((NG, 1)),
        out_shape=jax.ShapeDtypeStruct((NG, 1), f32),
        scratch_shapes=[pltpu.VMEM((NG, HID), f32),
                        pltpu.VMEM((NG, 1), f32)],
    )(agg2, z, deg2p, keptc, batchp, b3, W2, b2.reshape(1, 1))
    return out.reshape(-1)


# speedup vs baseline: 32.5722x; 1.0008x over previous
import dataclasses
import functools

import jax
import jax.numpy as jnp
from jax import lax
from jax.experimental import pallas as pl
from jax.experimental.pallas import tpu as pltpu
from jax.experimental.pallas import tpu_sc as plsc

N = 10000
E = 320000
D = 128
HID = 256
NG = 128
TOPK = 5000

NP = 10240
GR = NP
RS = NP + 128
NC, NS = 2, 16
NW = NC * NS
CHUNK = 10240
NB = CHUNK // 128
BB = 128
EP = NW * CHUNK

_mesh = plsc.VectorSubcoreMesh(core_axis_name="c", subcore_axis_name="s")

_cp_no_layout = pltpu.CompilerParams()
if "needs_layout_passes" in pltpu.CompilerParams.__dataclass_fields__:
    _cp_no_layout = dataclasses.replace(_cp_no_layout,
                                        needs_layout_passes=False)


def _zero16(dtype):
    return jnp.zeros((16,), dtype)


def _zero_1d(ref):
    z = _zero16(ref.dtype)

    @pl.loop(0, ref.shape[0] // 16)
    def _(i):
        ref[pl.ds(i * 16, 16)] = z


@functools.partial(
    pl.kernel,
    mesh=_mesh,
    compiler_params=_cp_no_layout,
    out_type=jax.ShapeDtypeStruct((NW, RS), jnp.float32),
    scratch_types=[
        pltpu.VMEM((RS,), jnp.float32),
        pltpu.VMEM((BB,), jnp.int32),
    ],
)
def _k1_deg(dst_hbm, deg_hbm, deg_v, dbuf):
    c = lax.axis_index("c")
    s = lax.axis_index("s")
    wid = s * NC + c
    _zero_1d(deg_v)
    ones16 = jnp.ones((16,), jnp.float32)
    base = wid * CHUNK

    @pl.loop(0, NB)
    def _(b):
        pltpu.sync_copy(dst_hbm.at[pl.ds(base + b * BB, BB)], dbuf)
        for j in range(BB // 16):
            d16 = dbuf[pl.ds(j * 16, 16)]
            plsc.addupdate_scatter(deg_v, [d16], ones16)

    pltpu.sync_copy(deg_v, deg_hbm.at[wid])


@functools.partial(
    pl.kernel,
    mesh=_mesh,
    out_type=jax.ShapeDtypeStruct((NC, NP, D), jnp.float32),
    scratch_types=[
        pltpu.VMEM((BB,), jnp.int32),
        pltpu.VMEM((BB,), jnp.int32),
        pltpu.VMEM((BB,), jnp.int32),
        pltpu.VMEM((BB,), jnp.int32),
        pltpu.VMEM((BB, D), jnp.float32),
        pltpu.VMEM((BB, D), jnp.float32),
        pltpu.VMEM((16, D), jnp.float32),
        pltpu.VMEM_SHARED((RS, D), jnp.float32),
        pltpu.SemaphoreType.DMA,
        pltpu.SemaphoreType.DMA,
        pltpu.SemaphoreType.DMA,
        pltpu.SemaphoreType.DMA,
    ],
)
def _k3_agg1(y_hbm, src_hbm, dst_hbm, zsl_hbm, agg_hbm, sidx0, sidx1,
             didx0, didx1, rows0, rows1, z_v, agg_sh, sem0, sem1,
             ssem0, ssem1):
    c = lax.axis_index("c")
    s = lax.axis_index("s")
    wid = s * NC + c
    pltpu.sync_copy(zsl_hbm, z_v)

    @pl.loop(s, RS // 16, step=NS)
    def _(j):
        pltpu.sync_copy(z_v, agg_sh.at[pl.ds(j * 16, 16)])

    plsc.subcore_barrier()
    base = wid * CHUNK
    sidx = (sidx0, sidx1)
    didx = (didx0, didx1)
    rows = (rows0, rows1)
    gsem = (sem0, sem1)
    ssem = (ssem0, ssem1)

    pltpu.sync_copy(src_hbm.at[pl.ds(base, BB)], sidx[0])
    pltpu.sync_copy(dst_hbm.at[pl.ds(base, BB)], didx[0])
    pltpu.async_copy(y_hbm.at[sidx[0]], rows[0], gsem[0])

    @pl.loop(0, NB, step=2)
    def _(b):
        for t in range(2):
            cur, nxt = t, 1 - t
            bb = b + t

            @pl.when(bb + 1 < NB)
            def _():
                @pl.when(bb + 1 >= 2)
                def _():
                    pltpu.make_async_copy(
                        rows[nxt], agg_sh.at[didx[nxt]], ssem[nxt]).wait()

                off = base + (bb + 1) * BB
                pltpu.sync_copy(src_hbm.at[pl.ds(off, BB)], sidx[nxt])
                pltpu.sync_copy(dst_hbm.at[pl.ds(off, BB)], didx[nxt])
                pltpu.async_copy(y_hbm.at[sidx[nxt]], rows[nxt], gsem[nxt])

            pltpu.make_async_copy(y_hbm.at[sidx[cur]], rows[cur],
                                  gsem[cur]).wait()
            pltpu.async_copy(rows[cur], agg_sh.at[didx[cur]], ssem[cur],
                             add=True)

    pltpu.make_async_copy(rows[0], agg_sh.at[didx[0]], ssem[0]).wait()
    pltpu.make_async_copy(rows[1], agg_sh.at[didx[1]], ssem[1]).wait()
    plsc.subcore_barrier()
    rows_per = NP // NS
    pltpu.sync_copy(agg_sh.at[pl.ds(s * rows_per, rows_per)],
                    agg_hbm.at[c].at[pl.ds(s * rows_per, rows_per)])


@functools.partial(
    pl.kernel,
    mesh=_mesh,
    compiler_params=_cp_no_layout,
    out_type=(
        jax.ShapeDtypeStruct((NW, CHUNK), jnp.int32),
        jax.ShapeDtypeStruct((NW, CHUNK), jnp.int32),
        jax.ShapeDtypeStruct((NW, 16), jnp.int32),
        jax.ShapeDtypeStruct((NW, RS), jnp.float32),
    ),
    scratch_types=[
        pltpu.VMEM((NP + 16,), jnp.int32),
        pltpu.VMEM((BB,), jnp.int32),
        pltpu.VMEM((BB,), jnp.int32),
        pltpu.VMEM((CHUNK,), jnp.int32),
        pltpu.VMEM((CHUNK,), jnp.int32),
        pltpu.VMEM((RS,), jnp.float32),
        pltpu.VMEM((16,), jnp.int32),
    ],
)
def _k5_compact(src_hbm, dst_hbm, kept_hbm, csrc_hbm, cdst_hbm, cnt_hbm,
                deg_hbm, kept_v, sbuf, dbuf, csrc_v, cdst_v, deg_v, cnt_v):
    c = lax.axis_index("c")
    s = lax.axis_index("s")
    wid = s * NC + c
    lane = lax.iota(jnp.int32, 16)
    _zero_1d(deg_v)
    pltpu.sync_copy(kept_hbm, kept_v.at[pl.ds(0, NP)])
    kept_v[pl.ds(NP, 16)] = _zero16(jnp.int32)
    zi = _zero16(jnp.int32)
    gi = jnp.full((16,), GR, jnp.int32)
    ones16 = jnp.ones((16,), jnp.float32)

    @pl.loop(0, CHUNK // 16)
    def _(i):
        csrc_v[pl.ds(i * 16, 16)] = zi
        cdst_v[pl.ds(i * 16, 16)] = gi

    base = wid * CHUNK

    def batch_body(b, cnt):
        pltpu.sync_copy(src_hbm.at[pl.ds(base + b * BB, BB)], sbuf)
        pltpu.sync_copy(dst_hbm.at[pl.ds(base + b * BB, BB)], dbuf)
        for j in range(BB // 16):
            s16 = sbuf[pl.ds(j * 16, 16)]
            d16 = dbuf[pl.ds(j * 16, 16)]
            ks = plsc.load_gather(kept_v, [s16])
            kd = plsc.load_gather(kept_v, [d16])
            vi = ks * kd
            valid = vi == 1
            pos = plsc.cumsum(vi)
            idx16 = cnt + pos - 1
            plsc.store_scatter(csrc_v, [idx16], s16, mask=valid)
            plsc.store_scatter(cdst_v, [idx16], d16, mask=valid)
            plsc.addupdate_scatter(deg_v, [jnp.where(valid, d16, GR)],
                                   ones16)
            cnt = cnt + jnp.sum(vi)
        return cnt

    total = lax.fori_loop(0, NB, batch_body, jnp.int32(0))
    cnt_v[...] = jnp.where(lane == 0, total, 0).astype(jnp.int32)
    pltpu.sync_copy(csrc_v, csrc_hbm.at[wid])
    pltpu.sync_copy(cdst_v, cdst_hbm.at[wid])
    pltpu.sync_copy(cnt_v, cnt_hbm.at[wid])
    pltpu.sync_copy(deg_v, deg_hbm.at[wid])


@functools.partial(
    pl.kernel,
    mesh=_mesh,
    out_type=jax.ShapeDtypeStruct((NC, NP, D), jnp.float32),
    scratch_types=[
        pltpu.VMEM((BB,), jnp.int32),
        pltpu.VMEM((BB,), jnp.int32),
        pltpu.VMEM((BB,), jnp.int32),
        pltpu.VMEM((BB,), jnp.int32),
        pltpu.VMEM((BB, D), jnp.float32),
        pltpu.VMEM((BB, D), jnp.float32),
        pltpu.VMEM((16, D), jnp.float32),
        pltpu.VMEM((16,), jnp.int32),
        pltpu.VMEM_SHARED((RS, D), jnp.float32),
        pltpu.SemaphoreType.DMA,
        pltpu.SemaphoreType.DMA,
    ],
)
def _k7_agg2(y2_hbm, csrc_hbm, cdst_hbm, cnt_hbm, zsl_hbm, agg_hbm,
             sidx0, sidx1, didx0, didx1, rows0, rows1, z_v, cnt_v,
             agg_sh, sem0, sem1):
    c = lax.axis_index("c")
    s = lax.axis_index("s")
    pltpu.sync_copy(zsl_hbm, z_v)

    @pl.loop(s, RS // 16, step=NS)
    def _(j):
        pltpu.sync_copy(z_v, agg_sh.at[pl.ds(j * 16, 16)])

    plsc.subcore_barrier()
    sidx = (sidx0, sidx1)
    didx = (didx0, didx1)
    rows = (rows0, rows1)
    sems = (sem0, sem1)

    for t in range(2):
        w = s * 2 + t
        pltpu.sync_copy(cnt_hbm.at[w], cnt_v)
        cw = cnt_v[...][0]
        npair = (cw + 255) // 256

        @pl.when(npair > 0)
        def _():
            pltpu.sync_copy(csrc_hbm.at[w, pl.ds(0, BB)], sidx[0])
            pltpu.sync_copy(cdst_hbm.at[w, pl.ds(0, BB)], didx[0])
            pltpu.async_copy(y2_hbm.at[c].at[sidx[0]], rows[0], sems[0])

            @pl.loop(0, npair)
            def _(p):
                for u in range(2):
                    bb = p * 2 + u
                    cur, nxt = u, 1 - u

                    @pl.when(bb + 1 < npair * 2)
                    def _():
                        off = (bb + 1) * BB
                        pltpu.sync_copy(csrc_hbm.at[w, pl.ds(off, BB)],
                                        sidx[nxt])
                        pltpu.sync_copy(cdst_hbm.at[w, pl.ds(off, BB)],
                                        didx[nxt])
                        pltpu.async_copy(y2_hbm.at[c].at[sidx[nxt]],
                                         rows[nxt], sems[nxt])

                    pltpu.make_async_copy(y2_hbm.at[c].at[sidx[cur]],
                                          rows[cur], sems[cur]).wait()
                    pltpu.sync_copy(rows[cur], agg_sh.at[didx[cur]],
                                    add=True)

    plsc.subcore_barrier()
    rows_per = NP // NS
    pltpu.sync_copy(agg_sh.at[pl.ds(s * rows_per, rows_per)],
                    agg_hbm.at[c].at[pl.ds(s * rows_per, rows_per)])


_RB = 1280
_GRID = NP // _RB


def _sum_partials(degp_blk):
    ones = jnp.ones((NW, 1), jnp.float32)
    return lax.dot_general(degp_blk, ones, (((0,), (0,)), ((), ())),
                           preferred_element_type=jnp.float32,
                           precision=lax.Precision.HIGHEST)


def _k2_body(degp_ref, x_ref, y_ref):
    dinv = lax.rsqrt(jnp.maximum(_sum_partials(degp_ref[...]) + 1.0, 1.0))
    y_ref[...] = dinv * x_ref[...]


def _k4a_body(x_ref, aggp_ref, degp_ref, w1_ref, b1_ref, p_ref, h1_ref,
              sc_ref):
    i = pl.program_id(0)
    dinv = lax.rsqrt(jnp.maximum(_sum_partials(degp_ref[...]) + 1.0, 1.0))
    u = dinv * (aggp_ref[0] + aggp_ref[1]) + (dinv * dinv) * x_ref[...]
    h1 = jnp.maximum(
        lax.dot_general(u, w1_ref[...], (((1,), (0,)), ((), ())),
                        preferred_element_type=jnp.float32,
                        precision=lax.Precision.HIGHEST)
        + b1_ref[...][None, :], 0.0)
    h1_ref[...] = h1
    p = p_ref[...]
    pn = jnp.sqrt(jnp.sum(p * p))
    sc = lax.dot_general(h1, p, (((1,), (0,)), ((), ())),
                         preferred_element_type=jnp.float32,
                         precision=lax.Precision.HIGHEST) / pn
    ridx = i * _RB + lax.broadcasted_iota(jnp.int32, (_RB, 1), 0)
    sc_ref[...] = jnp.where(ridx < N, sc, -jnp.inf)


def _k4b_body(sc_ref, kept_ref, gs_ref):
    s = sc_ref[...]
    bits = lax.bitcast_convert_type(s, jnp.int32)
    m = bits ^ jnp.where(bits < 0, jnp.int32(0x7FFFFFFF), jnp.int32(0))

    def bs(it, T):
        cand = T + (jnp.int32(1) << (jnp.int32(30) - it))
        c = jnp.sum((m >= cand).astype(jnp.int32))
        return jnp.where(c >= TOPK, cand, T)

    T = lax.fori_loop(0, 31, bs, jnp.int32(-2**31))
    cnt_gt = jnp.sum((m > T).astype(jnp.int32))
    need = (TOPK - cnt_gt).astype(jnp.float32)
    eq = (m == T)
    eqf = eq.astype(jnp.float32)
    r128 = lax.broadcasted_iota(jnp.int32, (128, 128), 0)
    c128 = lax.broadcasted_iota(jnp.int32, (128, 128), 1)
    u128 = (r128 < c128).astype(jnp.float32)
    p1 = lax.dot_general(eqf, u128, (((1,), (0,)), ((), ())),
                         preferred_element_type=jnp.float32,
                         precision=lax.Precision.HIGHEST)
    rowtot = jnp.sum(eqf, axis=1, keepdims=True)
    r80 = lax.broadcasted_iota(jnp.int32, (80, 80), 0)
    c80 = lax.broadcasted_iota(jnp.int32, (80, 80), 1)
    u80 = (r80 < c80).astype(jnp.float32)
    carry = lax.dot_general(u80, rowtot, (((1,), (0,)), ((), ())),
                            preferred_element_type=jnp.float32,
                            precision=lax.Precision.HIGHEST)
    prefix = p1 + carry
    kept = (m > T) | (eq & (prefix < need))
    keptf = kept.astype(jnp.float32)
    kept_ref[...] = keptf
    gs_ref[...] = keptf * jnp.tanh(s)


def _k4c_body(h1_ref, gs_ref, w3_ref, z_ref):
    xp = gs_ref[...] * h1_ref[...]
    z_ref[...] = lax.dot_general(xp, w3_ref[...], (((1,), (0,)), ((), ())),
                                 preferred_element_type=jnp.float32,
                                 precision=lax.Precision.HIGHEST)


def _dinv2_of(degp_blk, keptf):
    deg2 = _sum_partials(degp_blk) + keptf
    return keptf * lax.rsqrt(jnp.maximum(deg2, 1.0))


def _k6_body(z_ref, degp_ref, kept_ref, y2_ref):
    dinv2 = _dinv2_of(degp_ref[...], kept_ref[...])
    z = z_ref[...]
    y2_ref[0] = dinv2 * z[:, :D]
    y2_ref[1] = dinv2 * z[:, D:]


def _k8_body(agg2_ref, z_ref, degp_ref, kept_ref, batch_ref, b3_ref,
             w2_ref, b2_ref, out_ref, sums, cnt):
    i = pl.program_id(0)

    @pl.when(i == 0)
    def _():
        sums[...] = jnp.zeros_like(sums)
        cnt[...] = jnp.zeros_like(cnt)

    keptf = kept_ref[...]
    dinv2 = _dinv2_of(degp_ref[...], keptf)
    agg = jnp.concatenate([agg2_ref[0], agg2_ref[1]], axis=1)
    h3 = jnp.maximum(dinv2 * agg + (dinv2 * dinv2) * z_ref[...]
                     + b3_ref[...][None, :], 0.0)
    oh = (batch_ref[...] ==
          lax.broadcasted_iota(jnp.int32, (1, NG), 1)).astype(jnp.float32)
    wh3 = keptf * h3
    sums[...] += lax.dot_general(oh, wh3, (((0,), (0,)), ((), ())),
                                 preferred_element_type=jnp.float32,
                                 precision=lax.Precision.HIGHEST)
    cnt[...] += lax.dot_general(oh, keptf, (((0,), (0,)), ((), ())),
                                preferred_element_type=jnp.float32,
                                precision=lax.Precision.HIGHEST)

    @pl.when(i == _GRID - 1)
    def _():
        gm = sums[...] / jnp.maximum(cnt[...], 1.0)
        logit = lax.dot_general(gm, w2_ref[...], (((1,), (0,)), ((), ())),
                                preferred_element_type=jnp.float32,
                                precision=lax.Precision.HIGHEST)
        out_ref[...] = jax.nn.sigmoid(logit + b2_ref[...])


def _rows(block_cols):
    return pl.BlockSpec((_RB, block_cols), lambda i: (i, 0))


def _pair(block_cols):
    return pl.BlockSpec((2, _RB, block_cols), lambda i: (0, i, 0))


def _deg_spec():
    return pl.BlockSpec((NW, _RB), lambda i: (0, i))


def _full(shape):
    return pl.BlockSpec(shape, lambda i: tuple(0 for _ in shape))


def kernel(x, edge_list, dummy, batch, W1, b1, p_vec, W3, b3, W2, b2):
    f32 = jnp.float32
    src = edge_list[:, 0].astype(jnp.int32)
    dst = edge_list[:, 1].astype(jnp.int32)
    pad_e = CHUNK - E // NW
    srcp = jnp.concatenate(
        [src.reshape(NW, E // NW),
         jnp.zeros((NW, pad_e), jnp.int32)], axis=1).reshape(-1)
    dstp = jnp.concatenate(
        [dst.reshape(NW, E // NW),
         jnp.full((NW, pad_e), GR, jnp.int32)], axis=1).reshape(-1)
    xp = jnp.pad(x, ((0, NP - N), (0, 0)))
    batchp = jnp.pad(batch.astype(jnp.int32), (0, NP - N)).reshape(NP, 1)
    zsl = jnp.zeros((16, D), f32)

    deg1p = _k1_deg(dstp)

    y = pl.pallas_call(
        _k2_body,
        grid=(_GRID,),
        in_specs=[_deg_spec(), _rows(D)],
        out_specs=_rows(D),
        out_shape=jax.ShapeDtypeStruct((NP, D), f32),
    )(deg1p, xp)

    agg1p = _k3_agg1(y, srcp, dstp, zsl)

    h1, score = pl.pallas_call(
        _k4a_body,
        grid=(_GRID,),
        in_specs=[_rows(D), _pair(D), _deg_spec(), _full((D, HID)),
                  _full((HID,)), _full((HID, 1))],
        out_specs=[_rows(HID), _rows(1)],
        out_shape=[jax.ShapeDtypeStruct((NP, HID), f32),
                   jax.ShapeDtypeStruct((NP, 1), f32)],
    )(xp, agg1p, deg1p, W1, b1, p_vec.reshape(HID, 1))

    keptm, gsm = pl.pallas_call(
        _k4b_body,
        in_specs=[pl.BlockSpec((80, 128), lambda: (0, 0))],
        out_specs=[pl.BlockSpec((80, 128), lambda: (0, 0))] * 2,
        out_shape=[jax.ShapeDtypeStruct((80, 128), f32)] * 2,
    )(score.reshape(80, 128))
    keptc = keptm.reshape(NP, 1)
    kept_i = keptm.reshape(NP).astype(jnp.int32)

    z = pl.pallas_call(
        _k4c_body,
        grid=(_GRID,),
        in_specs=[_rows(HID), _rows(1), _full((HID, HID))],
        out_specs=_rows(HID),
        out_shape=jax.ShapeDtypeStruct((NP, HID), f32),
    )(h1, gsm.reshape(NP, 1), W3)

    csrc, cdst, cnts, deg2p = _k5_compact(srcp, dstp, kept_i)

    y2 = pl.pallas_call(
        _k6_body,
        grid=(_GRID,),
        in_specs=[_rows(HID), _deg_spec(), _rows(1)],
        out_specs=_pair(D),
        out_shape=jax.ShapeDtypeStruct((NC, NP, D), f32),
    )(z, deg2p, keptc)

    agg2 = _k7_agg2(y2, csrc, cdst, cnts, zsl)

    out = pl.pallas_call(
        _k8_body,
        grid=(_GRID,),
        in_specs=[_pair(D), _rows(HID), _deg_spec(), _rows(1),
                  pl.BlockSpec((_RB, 1), lambda i: (i, 0)),
                  _full((HID,)), _full((HID, 1)), _full((1, 1))],
        out_specs=_full((NG, 1)),
        out_shape=jax.ShapeDtypeStruct((NG, 1), f32),
        scratch_shapes=[pltpu.VMEM((NG, HID), f32),
                        pltpu.VMEM((NG, 1), f32)],
    )(agg2, z, deg2p, keptc, batchp, b3, W2, b2.reshape(1, 1))
    return out.reshape(-1)


# speedup vs baseline: 33.7890x; 1.0374x over previous
import dataclasses
import functools

import jax
import jax.numpy as jnp
from jax import lax
from jax.experimental import pallas as pl
from jax.experimental.pallas import tpu as pltpu
from jax.experimental.pallas import tpu_sc as plsc

N = 10000
E = 320000
D = 128
HID = 256
NG = 128
TOPK = 5000

NP = 10240
GR = NP
RS = NP + 128
NC, NS = 2, 16
NW = NC * NS
CHUNK = 10240
NB = CHUNK // 128
BB = 128
IBB = 10
IB = IBB * BB
EP = NW * CHUNK

_mesh = plsc.VectorSubcoreMesh(core_axis_name="c", subcore_axis_name="s")

_cp_no_layout = pltpu.CompilerParams()
if "needs_layout_passes" in pltpu.CompilerParams.__dataclass_fields__:
    _cp_no_layout = dataclasses.replace(_cp_no_layout,
                                        needs_layout_passes=False)


def _zero16(dtype):
    return jnp.zeros((16,), dtype)


def _zero_1d(ref):
    z = _zero16(ref.dtype)

    @pl.loop(0, ref.shape[0] // 16)
    def _(i):
        ref[pl.ds(i * 16, 16)] = z


def _copy128_local(src_ref, src_off, dst_ref):
    for j in range(BB // 16):
        dst_ref[pl.ds(j * 16, 16)] = src_ref[pl.ds(src_off + j * 16, 16)]


@functools.partial(
    pl.kernel,
    mesh=_mesh,
    compiler_params=_cp_no_layout,
    out_type=jax.ShapeDtypeStruct((NW, RS), jnp.float32),
    scratch_types=[
        pltpu.VMEM((RS,), jnp.float32),
        pltpu.VMEM((BB,), jnp.int32),
    ],
)
def _k1_deg(dst_hbm, deg_hbm, deg_v, dbuf):
    c = lax.axis_index("c")
    s = lax.axis_index("s")
    wid = s * NC + c
    _zero_1d(deg_v)
    ones16 = jnp.ones((16,), jnp.float32)
    base = wid * CHUNK

    @pl.loop(0, NB)
    def _(b):
        pltpu.sync_copy(dst_hbm.at[pl.ds(base + b * BB, BB)], dbuf)
        for j in range(BB // 16):
            d16 = dbuf[pl.ds(j * 16, 16)]
            plsc.addupdate_scatter(deg_v, [d16], ones16)

    pltpu.sync_copy(deg_v, deg_hbm.at[wid])


@functools.partial(
    pl.kernel,
    mesh=_mesh,
    out_type=jax.ShapeDtypeStruct((NC, NP, D), jnp.float32),
    scratch_types=[
        pltpu.VMEM((BB,), jnp.int32),
        pltpu.VMEM((BB,), jnp.int32),
        pltpu.VMEM((BB,), jnp.int32),
        pltpu.VMEM((BB,), jnp.int32),
        pltpu.VMEM((IB,), jnp.int32),
        pltpu.VMEM((IB,), jnp.int32),
        pltpu.VMEM((BB, D), jnp.float32),
        pltpu.VMEM((BB, D), jnp.float32),
        pltpu.VMEM((16, D), jnp.float32),
        pltpu.VMEM_SHARED((RS, D), jnp.float32),
        pltpu.SemaphoreType.DMA,
        pltpu.SemaphoreType.DMA,
        pltpu.SemaphoreType.DMA,
        pltpu.SemaphoreType.DMA,
    ],
)
def _k3_agg1(y_hbm, src_hbm, dst_hbm, zsl_hbm, agg_hbm, sidx0, sidx1,
             didx0, didx1, isrc, idst, rows0, rows1, z_v, agg_sh, sem0, sem1,
             ssem0, ssem1):
    c = lax.axis_index("c")
    s = lax.axis_index("s")
    wid = s * NC + c
    pltpu.sync_copy(zsl_hbm, z_v)

    @pl.loop(s, RS // 16, step=NS)
    def _(j):
        pltpu.sync_copy(z_v, agg_sh.at[pl.ds(j * 16, 16)])

    plsc.subcore_barrier()
    base = wid * CHUNK
    sidx = (sidx0, sidx1)
    didx = (didx0, didx1)
    rows = (rows0, rows1)
    gsem = (sem0, sem1)
    ssem = (ssem0, ssem1)

    pltpu.sync_copy(src_hbm.at[pl.ds(base, IB)], isrc)
    pltpu.sync_copy(dst_hbm.at[pl.ds(base, IB)], idst)
    _copy128_local(isrc, 0, sidx[0])
    _copy128_local(idst, 0, didx[0])
    pltpu.async_copy(y_hbm.at[sidx[0]], rows[0], gsem[0])

    @pl.loop(0, NB, step=2)
    def _(b):
        for t in range(2):
            cur, nxt = t, 1 - t
            bb = b + t

            @pl.when(bb + 1 < NB)
            def _():
                @pl.when(bb + 1 >= 2)
                def _():
                    pltpu.make_async_copy(
                        rows[nxt], agg_sh.at[didx[nxt]], ssem[nxt]).wait()

                nb1 = bb + 1

                @pl.when(nb1 % IBB == 0)
                def _():
                    off = base + nb1 * BB
                    pltpu.sync_copy(src_hbm.at[pl.ds(off, IB)], isrc)
                    pltpu.sync_copy(dst_hbm.at[pl.ds(off, IB)], idst)

                blk = (nb1 % IBB) * BB
                _copy128_local(isrc, blk, sidx[nxt])
                _copy128_local(idst, blk, didx[nxt])
                pltpu.async_copy(y_hbm.at[sidx[nxt]], rows[nxt], gsem[nxt])

            pltpu.make_async_copy(y_hbm.at[sidx[cur]], rows[cur],
                                  gsem[cur]).wait()
            pltpu.async_copy(rows[cur], agg_sh.at[didx[cur]], ssem[cur],
                             add=True)

    pltpu.make_async_copy(rows[0], agg_sh.at[didx[0]], ssem[0]).wait()
    pltpu.make_async_copy(rows[1], agg_sh.at[didx[1]], ssem[1]).wait()
    plsc.subcore_barrier()
    rows_per = NP // NS
    pltpu.sync_copy(agg_sh.at[pl.ds(s * rows_per, rows_per)],
                    agg_hbm.at[c].at[pl.ds(s * rows_per, rows_per)])


@functools.partial(
    pl.kernel,
    mesh=_mesh,
    compiler_params=_cp_no_layout,
    out_type=(
        jax.ShapeDtypeStruct((NW, CHUNK), jnp.int32),
        jax.ShapeDtypeStruct((NW, CHUNK), jnp.int32),
        jax.ShapeDtypeStruct((NW, 16), jnp.int32),
        jax.ShapeDtypeStruct((NW, RS), jnp.float32),
    ),
    scratch_types=[
        pltpu.VMEM((NP + 16,), jnp.int32),
        pltpu.VMEM((BB,), jnp.int32),
        pltpu.VMEM((BB,), jnp.int32),
        pltpu.VMEM((CHUNK,), jnp.int32),
        pltpu.VMEM((CHUNK,), jnp.int32),
        pltpu.VMEM((RS,), jnp.float32),
        pltpu.VMEM((16,), jnp.int32),
    ],
)
def _k5_compact(src_hbm, dst_hbm, kept_hbm, csrc_hbm, cdst_hbm, cnt_hbm,
                deg_hbm, kept_v, sbuf, dbuf, csrc_v, cdst_v, deg_v, cnt_v):
    c = lax.axis_index("c")
    s = lax.axis_index("s")
    wid = s * NC + c
    lane = lax.iota(jnp.int32, 16)
    _zero_1d(deg_v)
    pltpu.sync_copy(kept_hbm, kept_v.at[pl.ds(0, NP)])
    kept_v[pl.ds(NP, 16)] = _zero16(jnp.int32)
    zi = _zero16(jnp.int32)
    gi = jnp.full((16,), GR, jnp.int32)
    ones16 = jnp.ones((16,), jnp.float32)

    @pl.loop(0, CHUNK // 16)
    def _(i):
        csrc_v[pl.ds(i * 16, 16)] = zi
        cdst_v[pl.ds(i * 16, 16)] = gi

    base = wid * CHUNK

    def batch_body(b, cnt):
        pltpu.sync_copy(src_hbm.at[pl.ds(base + b * BB, BB)], sbuf)
        pltpu.sync_copy(dst_hbm.at[pl.ds(base + b * BB, BB)], dbuf)
        for j in range(BB // 16):
            s16 = sbuf[pl.ds(j * 16, 16)]
            d16 = dbuf[pl.ds(j * 16, 16)]
            ks = plsc.load_gather(kept_v, [s16])
            kd = plsc.load_gather(kept_v, [d16])
            vi = ks * kd
            valid = vi == 1
            pos = plsc.cumsum(vi)
            idx16 = cnt + pos - 1
            plsc.store_scatter(csrc_v, [idx16], s16, mask=valid)
            plsc.store_scatter(cdst_v, [idx16], d16, mask=valid)
            plsc.addupdate_scatter(deg_v, [jnp.where(valid, d16, GR)],
                                   ones16)
            cnt = cnt + jnp.sum(vi)
        return cnt

    total = lax.fori_loop(0, NB, batch_body, jnp.int32(0))
    cnt_v[...] = jnp.where(lane == 0, total, 0).astype(jnp.int32)
    pltpu.sync_copy(csrc_v, csrc_hbm.at[wid])
    pltpu.sync_copy(cdst_v, cdst_hbm.at[wid])
    pltpu.sync_copy(cnt_v, cnt_hbm.at[wid])
    pltpu.sync_copy(deg_v, deg_hbm.at[wid])


@functools.partial(
    pl.kernel,
    mesh=_mesh,
    out_type=jax.ShapeDtypeStruct((NC, NP, D), jnp.float32),
    scratch_types=[
        pltpu.VMEM((BB,), jnp.int32),
        pltpu.VMEM((BB,), jnp.int32),
        pltpu.VMEM((BB,), jnp.int32),
        pltpu.VMEM((BB,), jnp.int32),
        pltpu.VMEM((BB, D), jnp.float32),
        pltpu.VMEM((BB, D), jnp.float32),
        pltpu.VMEM((16, D), jnp.float32),
        pltpu.VMEM((16,), jnp.int32),
        pltpu.VMEM_SHARED((RS, D), jnp.float32),
        pltpu.SemaphoreType.DMA,
        pltpu.SemaphoreType.DMA,
    ],
)
def _k7_agg2(y2_hbm, csrc_hbm, cdst_hbm, cnt_hbm, zsl_hbm, agg_hbm,
             sidx0, sidx1, didx0, didx1, rows0, rows1, z_v, cnt_v,
             agg_sh, sem0, sem1):
    c = lax.axis_index("c")
    s = lax.axis_index("s")
    pltpu.sync_copy(zsl_hbm, z_v)

    @pl.loop(s, RS // 16, step=NS)
    def _(j):
        pltpu.sync_copy(z_v, agg_sh.at[pl.ds(j * 16, 16)])

    plsc.subcore_barrier()
    sidx = (sidx0, sidx1)
    didx = (didx0, didx1)
    rows = (rows0, rows1)
    sems = (sem0, sem1)

    for t in range(2):
        w = s * 2 + t
        pltpu.sync_copy(cnt_hbm.at[w], cnt_v)
        cw = cnt_v[...][0]
        npair = (cw + 255) // 256

        @pl.when(npair > 0)
        def _():
            pltpu.sync_copy(csrc_hbm.at[w, pl.ds(0, BB)], sidx[0])
            pltpu.sync_copy(cdst_hbm.at[w, pl.ds(0, BB)], didx[0])
            pltpu.async_copy(y2_hbm.at[c].at[sidx[0]], rows[0], sems[0])

            @pl.loop(0, npair)
            def _(p):
                for u in range(2):
                    bb = p * 2 + u
                    cur, nxt = u, 1 - u

                    @pl.when(bb + 1 < npair * 2)
                    def _():
                        off = (bb + 1) * BB
                        pltpu.sync_copy(csrc_hbm.at[w, pl.ds(off, BB)],
                                        sidx[nxt])
                        pltpu.sync_copy(cdst_hbm.at[w, pl.ds(off, BB)],
                                        didx[nxt])
                        pltpu.async_copy(y2_hbm.at[c].at[sidx[nxt]],
                                         rows[nxt], sems[nxt])

                    pltpu.make_async_copy(y2_hbm.at[c].at[sidx[cur]],
                                          rows[cur], sems[cur]).wait()
                    pltpu.sync_copy(rows[cur], agg_sh.at[didx[cur]],
                                    add=True)

    plsc.subcore_barrier()
    rows_per = NP // NS
    pltpu.sync_copy(agg_sh.at[pl.ds(s * rows_per, rows_per)],
                    agg_hbm.at[c].at[pl.ds(s * rows_per, rows_per)])


_RB = 1280
_GRID = NP // _RB


def _sum_partials(degp_blk):
    ones = jnp.ones((NW, 1), jnp.float32)
    return lax.dot_general(degp_blk, ones, (((0,), (0,)), ((), ())),
                           preferred_element_type=jnp.float32,
                           precision=lax.Precision.HIGHEST)


def _k2_body(degp_ref, x_ref, y_ref):
    dinv = lax.rsqrt(jnp.maximum(_sum_partials(degp_ref[...]) + 1.0, 1.0))
    y_ref[...] = dinv * x_ref[...]


def _k4a_body(x_ref, aggp_ref, degp_ref, w1_ref, b1_ref, p_ref, h1_ref,
              sc_ref):
    i = pl.program_id(0)
    dinv = lax.rsqrt(jnp.maximum(_sum_partials(degp_ref[...]) + 1.0, 1.0))
    u = dinv * (aggp_ref[0] + aggp_ref[1]) + (dinv * dinv) * x_ref[...]
    h1 = jnp.maximum(
        lax.dot_general(u, w1_ref[...], (((1,), (0,)), ((), ())),
                        preferred_element_type=jnp.float32,
                        precision=lax.Precision.HIGHEST)
        + b1_ref[...][None, :], 0.0)
    h1_ref[...] = h1
    p = p_ref[...]
    pn = jnp.sqrt(jnp.sum(p * p))
    sc = lax.dot_general(h1, p, (((1,), (0,)), ((), ())),
                         preferred_element_type=jnp.float32,
                         precision=lax.Precision.HIGHEST) / pn
    ridx = i * _RB + lax.broadcasted_iota(jnp.int32, (_RB, 1), 0)
    sc_ref[...] = jnp.where(ridx < N, sc, -jnp.inf)


def _k4b_body(sc_ref, kept_ref, gs_ref):
    s = sc_ref[...]
    bits = lax.bitcast_convert_type(s, jnp.int32)
    m = bits ^ jnp.where(bits < 0, jnp.int32(0x7FFFFFFF), jnp.int32(0))

    def bs(it, T):
        cand = T + (jnp.int32(1) << (jnp.int32(30) - it))
        c = jnp.sum((m >= cand).astype(jnp.int32))
        return jnp.where(c >= TOPK, cand, T)

    T = lax.fori_loop(0, 31, bs, jnp.int32(-2**31))
    cnt_gt = jnp.sum((m > T).astype(jnp.int32))
    need = (TOPK - cnt_gt).astype(jnp.float32)
    eq = (m == T)
    eqf = eq.astype(jnp.float32)
    r128 = lax.broadcasted_iota(jnp.int32, (128, 128), 0)
    c128 = lax.broadcasted_iota(jnp.int32, (128, 128), 1)
    u128 = (r128 < c128).astype(jnp.float32)
    p1 = lax.dot_general(eqf, u128, (((1,), (0,)), ((), ())),
                         preferred_element_type=jnp.float32,
                         precision=lax.Precision.HIGHEST)
    rowtot = jnp.sum(eqf, axis=1, keepdims=True)
    r80 = lax.broadcasted_iota(jnp.int32, (80, 80), 0)
    c80 = lax.broadcasted_iota(jnp.int32, (80, 80), 1)
    u80 = (r80 < c80).astype(jnp.float32)
    carry = lax.dot_general(u80, rowtot, (((1,), (0,)), ((), ())),
                            preferred_element_type=jnp.float32,
                            precision=lax.Precision.HIGHEST)
    prefix = p1 + carry
    kept = (m > T) | (eq & (prefix < need))
    keptf = kept.astype(jnp.float32)
    kept_ref[...] = keptf
    gs_ref[...] = keptf * jnp.tanh(s)


def _k4c_body(h1_ref, gs_ref, w3_ref, z_ref):
    xp = gs_ref[...] * h1_ref[...]
    z_ref[...] = lax.dot_general(xp, w3_ref[...], (((1,), (0,)), ((), ())),
                                 preferred_element_type=jnp.float32,
                                 precision=lax.Precision.HIGHEST)


def _dinv2_of(degp_blk, keptf):
    deg2 = _sum_partials(degp_blk) + keptf
    return keptf * lax.rsqrt(jnp.maximum(deg2, 1.0))


def _k6_body(z_ref, degp_ref, kept_ref, y2_ref):
    dinv2 = _dinv2_of(degp_ref[...], kept_ref[...])
    z = z_ref[...]
    y2_ref[0] = dinv2 * z[:, :D]
    y2_ref[1] = dinv2 * z[:, D:]


def _k8_body(agg2_ref, z_ref, degp_ref, kept_ref, batch_ref, b3_ref,
             w2_ref, b2_ref, out_ref, sums, cnt):
    i = pl.program_id(0)

    @pl.when(i == 0)
    def _():
        sums[...] = jnp.zeros_like(sums)
        cnt[...] = jnp.zeros_like(cnt)

    keptf = kept_ref[...]
    dinv2 = _dinv2_of(degp_ref[...], keptf)
    agg = jnp.concatenate([agg2_ref[0], agg2_ref[1]], axis=1)
    h3 = jnp.maximum(dinv2 * agg + (dinv2 * dinv2) * z_ref[...]
                     + b3_ref[...][None, :], 0.0)
    oh = (batch_ref[...] ==
          lax.broadcasted_iota(jnp.int32, (1, NG), 1)).astype(jnp.float32)
    wh3 = keptf * h3
    sums[...] += lax.dot_general(oh, wh3, (((0,), (0,)), ((), ())),
                                 preferred_element_type=jnp.float32,
                                 precision=lax.Precision.HIGHEST)
    cnt[...] += lax.dot_general(oh, keptf, (((0,), (0,)), ((), ())),
                                preferred_element_type=jnp.float32,
                                precision=lax.Precision.HIGHEST)

    @pl.when(i == _GRID - 1)
    def _():
        gm = sums[...] / jnp.maximum(cnt[...], 1.0)
        logit = lax.dot_general(gm, w2_ref[...], (((1,), (0,)), ((), ())),
                                preferred_element_type=jnp.float32,
                                precision=lax.Precision.HIGHEST)
        out_ref[...] = jax.nn.sigmoid(logit + b2_ref[...])


def _rows(block_cols):
    return pl.BlockSpec((_RB, block_cols), lambda i: (i, 0))


def _pair(block_cols):
    return pl.BlockSpec((2, _RB, block_cols), lambda i: (0, i, 0))


def _deg_spec():
    return pl.BlockSpec((NW, _RB), lambda i: (0, i))


def _full(shape):
    return pl.BlockSpec(shape, lambda i: tuple(0 for _ in shape))


def kernel(x, edge_list, dummy, batch, W1, b1, p_vec, W3, b3, W2, b2):
    f32 = jnp.float32
    src = edge_list[:, 0].astype(jnp.int32)
    dst = edge_list[:, 1].astype(jnp.int32)
    pad_e = CHUNK - E // NW
    srcp = jnp.concatenate(
        [src.reshape(NW, E // NW),
         jnp.zeros((NW, pad_e), jnp.int32)], axis=1).reshape(-1)
    dstp = jnp.concatenate(
        [dst.reshape(NW, E // NW),
         jnp.full((NW, pad_e), GR, jnp.int32)], axis=1).reshape(-1)
    xp = jnp.pad(x, ((0, NP - N), (0, 0)))
    batchp = jnp.pad(batch.astype(jnp.int32), (0, NP - N)).reshape(NP, 1)
    zsl = jnp.zeros((16, D), f32)

    deg1p = _k1_deg(dstp)

    y = pl.pallas_call(
        _k2_body,
        grid=(_GRID,),
        in_specs=[_deg_spec(), _rows(D)],
        out_specs=_rows(D),
        out_shape=jax.ShapeDtypeStruct((NP, D), f32),
    )(deg1p, xp)

    agg1p = _k3_agg1(y, srcp, dstp, zsl)

    h1, score = pl.pallas_call(
        _k4a_body,
        grid=(_GRID,),
        in_specs=[_rows(D), _pair(D), _deg_spec(), _full((D, HID)),
                  _full((HID,)), _full((HID, 1))],
        out_specs=[_rows(HID), _rows(1)],
        out_shape=[jax.ShapeDtypeStruct((NP, HID), f32),
                   jax.ShapeDtypeStruct((NP, 1), f32)],
    )(xp, agg1p, deg1p, W1, b1, p_vec.reshape(HID, 1))

    keptm, gsm = pl.pallas_call(
        _k4b_body,
        in_specs=[pl.BlockSpec((80, 128), lambda: (0, 0))],
        out_specs=[pl.BlockSpec((80, 128), lambda: (0, 0))] * 2,
        out_shape=[jax.ShapeDtypeStruct((80, 128), f32)] * 2,
    )(score.reshape(80, 128))
    keptc = keptm.reshape(NP, 1)
    kept_i = keptm.reshape(NP).astype(jnp.int32)

    z = pl.pallas_call(
        _k4c_body,
        grid=(_GRID,),
        in_specs=[_rows(HID), _rows(1), _full((HID, HID))],
        out_specs=_rows(HID),
        out_shape=jax.ShapeDtypeStruct((NP, HID), f32),
    )(h1, gsm.reshape(NP, 1), W3)

    csrc, cdst, cnts, deg2p = _k5_compact(srcp, dstp, kept_i)

    y2 = pl.pallas_call(
        _k6_body,
        grid=(_GRID,),
        in_specs=[_rows(HID), _deg_spec(), _rows(1)],
        out_specs=_pair(D),
        out_shape=jax.ShapeDtypeStruct((NC, NP, D), f32),
    )(z, deg2p, keptc)

    agg2 = _k7_agg2(y2, csrc, cdst, cnts, zsl)

    out = pl.pallas_call(
        _k8_body,
        grid=(_GRID,),
        in_specs=[_pair(D), _rows(HID), _deg_spec(), _rows(1),
                  pl.BlockSpec((_RB, 1), lambda i: (i, 0)),
                  _full((HID,)), _full((HID, 1)), _full((1, 1))],
        out_specs=_full((NG, 1)),
        out_shape=jax.ShapeDtypeStruct((NG, 1), f32),
        scratch_shapes=[pltpu.VMEM((NG, HID), f32),
                        pltpu.VMEM((NG, 1), f32)],
    )(agg2, z, deg2p, keptc, batchp, b3, W2, b2.reshape(1, 1))
    return out.reshape(-1)


# speedup vs baseline: 33.7930x; 1.0001x over previous
import dataclasses
import functools

import jax
import jax.numpy as jnp
from jax import lax
from jax.experimental import pallas as pl
from jax.experimental.pallas import tpu as pltpu
from jax.experimental.pallas import tpu_sc as plsc

N = 10000
E = 320000
D = 128
HID = 256
NG = 128
TOPK = 5000

NP = 10240
GR = NP
RS = NP + 128
NC, NS = 2, 16
NW = NC * NS
CHUNK = 10240
NB = CHUNK // 128
BB = 128
IBB = 10
IB = IBB * BB
EP = NW * CHUNK

_mesh = plsc.VectorSubcoreMesh(core_axis_name="c", subcore_axis_name="s")

_cp_no_layout = pltpu.CompilerParams()
if "needs_layout_passes" in pltpu.CompilerParams.__dataclass_fields__:
    _cp_no_layout = dataclasses.replace(_cp_no_layout,
                                        needs_layout_passes=False)


def _zero16(dtype):
    return jnp.zeros((16,), dtype)


def _zero_1d(ref):
    z = _zero16(ref.dtype)

    @pl.loop(0, ref.shape[0] // 16)
    def _(i):
        ref[pl.ds(i * 16, 16)] = z


def _copy128_local(src_ref, src_off, dst_ref):
    for j in range(BB // 16):
        dst_ref[pl.ds(j * 16, 16)] = src_ref[pl.ds(src_off + j * 16, 16)]


@functools.partial(
    pl.kernel,
    mesh=_mesh,
    compiler_params=_cp_no_layout,
    out_type=jax.ShapeDtypeStruct((NW, RS), jnp.float32),
    scratch_types=[
        pltpu.VMEM((RS,), jnp.float32),
        pltpu.VMEM((BB,), jnp.int32),
    ],
)
def _k1_deg(dst_hbm, deg_hbm, deg_v, dbuf):
    c = lax.axis_index("c")
    s = lax.axis_index("s")
    wid = s * NC + c
    _zero_1d(deg_v)
    ones16 = jnp.ones((16,), jnp.float32)
    base = wid * CHUNK

    @pl.loop(0, NB)
    def _(b):
        pltpu.sync_copy(dst_hbm.at[pl.ds(base + b * BB, BB)], dbuf)
        for j in range(BB // 16):
            d16 = dbuf[pl.ds(j * 16, 16)]
            plsc.addupdate_scatter(deg_v, [d16], ones16)

    pltpu.sync_copy(deg_v, deg_hbm.at[wid])


@functools.partial(
    pl.kernel,
    mesh=_mesh,
    out_type=jax.ShapeDtypeStruct((NC, NP, D), jnp.float32),
    scratch_types=[
        pltpu.VMEM((BB,), jnp.int32),
        pltpu.VMEM((BB,), jnp.int32),
        pltpu.VMEM((BB,), jnp.int32),
        pltpu.VMEM((BB,), jnp.int32),
        pltpu.VMEM((IB,), jnp.int32),
        pltpu.VMEM((IB,), jnp.int32),
        pltpu.VMEM((BB, D), jnp.float32),
        pltpu.VMEM((BB, D), jnp.float32),
        pltpu.VMEM((16, D), jnp.float32),
        pltpu.VMEM_SHARED((RS, D), jnp.float32),
        pltpu.SemaphoreType.DMA,
        pltpu.SemaphoreType.DMA,
        pltpu.SemaphoreType.DMA,
        pltpu.SemaphoreType.DMA,
    ],
)
def _k3_agg1(y_hbm, src_hbm, dst_hbm, zsl_hbm, agg_hbm, sidx0, sidx1,
             didx0, didx1, isrc, idst, rows0, rows1, z_v, agg_sh, sem0, sem1,
             ssem0, ssem1):
    c = lax.axis_index("c")
    s = lax.axis_index("s")
    wid = s * NC + c
    pltpu.sync_copy(zsl_hbm, z_v)

    @pl.loop(s, RS // 16, step=NS)
    def _(j):
        pltpu.sync_copy(z_v, agg_sh.at[pl.ds(j * 16, 16)])

    plsc.subcore_barrier()
    base = wid * CHUNK
    sidx = (sidx0, sidx1)
    didx = (didx0, didx1)
    rows = (rows0, rows1)
    gsem = (sem0, sem1)
    ssem = (ssem0, ssem1)

    pltpu.sync_copy(src_hbm.at[pl.ds(base, IB)], isrc)
    pltpu.sync_copy(dst_hbm.at[pl.ds(base, IB)], idst)
    _copy128_local(isrc, 0, sidx[0])
    _copy128_local(idst, 0, didx[0])
    pltpu.async_copy(y_hbm.at[sidx[0]], rows[0], gsem[0])

    @pl.loop(0, NB, step=2)
    def _(b):
        for t in range(2):
            cur, nxt = t, 1 - t
            bb = b + t

            @pl.when(bb + 1 < NB)
            def _():
                @pl.when(bb + 1 >= 2)
                def _():
                    pltpu.make_async_copy(
                        rows[nxt], agg_sh.at[didx[nxt]], ssem[nxt]).wait()

                nb1 = bb + 1

                @pl.when(nb1 % IBB == 0)
                def _():
                    off = base + nb1 * BB
                    pltpu.sync_copy(src_hbm.at[pl.ds(off, IB)], isrc)
                    pltpu.sync_copy(dst_hbm.at[pl.ds(off, IB)], idst)

                blk = (nb1 % IBB) * BB
                _copy128_local(isrc, blk, sidx[nxt])
                _copy128_local(idst, blk, didx[nxt])
                pltpu.async_copy(y_hbm.at[sidx[nxt]], rows[nxt], gsem[nxt])

            pltpu.make_async_copy(y_hbm.at[sidx[cur]], rows[cur],
                                  gsem[cur]).wait()
            pltpu.async_copy(rows[cur], agg_sh.at[didx[cur]], ssem[cur],
                             add=True)

    pltpu.make_async_copy(rows[0], agg_sh.at[didx[0]], ssem[0]).wait()
    pltpu.make_async_copy(rows[1], agg_sh.at[didx[1]], ssem[1]).wait()
    plsc.subcore_barrier()
    rows_per = NP // NS
    pltpu.sync_copy(agg_sh.at[pl.ds(s * rows_per, rows_per)],
                    agg_hbm.at[c].at[pl.ds(s * rows_per, rows_per)])


@functools.partial(
    pl.kernel,
    mesh=_mesh,
    compiler_params=_cp_no_layout,
    out_type=(
        jax.ShapeDtypeStruct((NW, CHUNK), jnp.int32),
        jax.ShapeDtypeStruct((NW, CHUNK), jnp.int32),
        jax.ShapeDtypeStruct((NW, 16), jnp.int32),
        jax.ShapeDtypeStruct((NW, RS), jnp.float32),
    ),
    scratch_types=[
        pltpu.VMEM((NP + 16,), jnp.int32),
        pltpu.VMEM((BB,), jnp.int32),
        pltpu.VMEM((BB,), jnp.int32),
        pltpu.VMEM((CHUNK,), jnp.int32),
        pltpu.VMEM((CHUNK,), jnp.int32),
        pltpu.VMEM((RS,), jnp.float32),
        pltpu.VMEM((16,), jnp.int32),
    ],
)
def _k5_compact(src_hbm, dst_hbm, kept_hbm, csrc_hbm, cdst_hbm, cnt_hbm,
                deg_hbm, kept_v, sbuf, dbuf, csrc_v, cdst_v, deg_v, cnt_v):
    c = lax.axis_index("c")
    s = lax.axis_index("s")
    wid = s * NC + c
    lane = lax.iota(jnp.int32, 16)
    _zero_1d(deg_v)
    pltpu.sync_copy(kept_hbm, kept_v.at[pl.ds(0, NP)])
    kept_v[pl.ds(NP, 16)] = _zero16(jnp.int32)
    zi = _zero16(jnp.int32)
    gi = jnp.full((16,), GR, jnp.int32)
    ones16 = jnp.ones((16,), jnp.float32)

    @pl.loop(0, CHUNK // 16)
    def _(i):
        csrc_v[pl.ds(i * 16, 16)] = zi
        cdst_v[pl.ds(i * 16, 16)] = gi

    base = wid * CHUNK

    def batch_body(b, cnt):
        pltpu.sync_copy(src_hbm.at[pl.ds(base + b * BB, BB)], sbuf)
        pltpu.sync_copy(dst_hbm.at[pl.ds(base + b * BB, BB)], dbuf)
        for j in range(BB // 16):
            s16 = sbuf[pl.ds(j * 16, 16)]
            d16 = dbuf[pl.ds(j * 16, 16)]
            ks = plsc.load_gather(kept_v, [s16])
            kd = plsc.load_gather(kept_v, [d16])
            vi = ks * kd
            valid = vi == 1
            pos = plsc.cumsum(vi)
            idx16 = cnt + pos - 1
            plsc.store_scatter(csrc_v, [idx16], s16, mask=valid)
            plsc.store_scatter(cdst_v, [idx16], d16, mask=valid)
            plsc.addupdate_scatter(deg_v, [jnp.where(valid, d16, GR)],
                                   ones16)
            cnt = cnt + jnp.sum(vi)
        return cnt

    total = lax.fori_loop(0, NB, batch_body, jnp.int32(0))
    cnt_v[...] = jnp.where(lane == 0, total, 0).astype(jnp.int32)
    pltpu.sync_copy(csrc_v, csrc_hbm.at[wid])
    pltpu.sync_copy(cdst_v, cdst_hbm.at[wid])
    pltpu.sync_copy(cnt_v, cnt_hbm.at[wid])
    pltpu.sync_copy(deg_v, deg_hbm.at[wid])


@functools.partial(
    pl.kernel,
    mesh=_mesh,
    out_type=jax.ShapeDtypeStruct((NC, NP, D), jnp.float32),
    scratch_types=[
        pltpu.VMEM((BB,), jnp.int32),
        pltpu.VMEM((BB,), jnp.int32),
        pltpu.VMEM((BB,), jnp.int32),
        pltpu.VMEM((BB,), jnp.int32),
        pltpu.VMEM((BB, D), jnp.float32),
        pltpu.VMEM((BB, D), jnp.float32),
        pltpu.VMEM((16, D), jnp.float32),
        pltpu.VMEM((16,), jnp.int32),
        pltpu.VMEM_SHARED((RS, D), jnp.float32),
        pltpu.SemaphoreType.DMA,
        pltpu.SemaphoreType.DMA,
        pltpu.SemaphoreType.DMA,
        pltpu.SemaphoreType.DMA,
    ],
)
def _k7_agg2(y2_hbm, csrc_hbm, cdst_hbm, cnt_hbm, zsl_hbm, agg_hbm,
             sidx0, sidx1, didx0, didx1, rows0, rows1, z_v, cnt_v,
             agg_sh, sem0, sem1, ssem0, ssem1):
    c = lax.axis_index("c")
    s = lax.axis_index("s")
    pltpu.sync_copy(zsl_hbm, z_v)

    @pl.loop(s, RS // 16, step=NS)
    def _(j):
        pltpu.sync_copy(z_v, agg_sh.at[pl.ds(j * 16, 16)])

    plsc.subcore_barrier()
    sidx = (sidx0, sidx1)
    didx = (didx0, didx1)
    rows = (rows0, rows1)
    sems = (sem0, sem1)
    ssem = (ssem0, ssem1)

    for t in range(2):
        w = s * 2 + t
        pltpu.sync_copy(cnt_hbm.at[w], cnt_v)
        cw = cnt_v[...][0]
        npair = (cw + 255) // 256

        @pl.when(npair > 0)
        def _():
            pltpu.sync_copy(csrc_hbm.at[w, pl.ds(0, BB)], sidx[0])
            pltpu.sync_copy(cdst_hbm.at[w, pl.ds(0, BB)], didx[0])
            pltpu.async_copy(y2_hbm.at[c].at[sidx[0]], rows[0], sems[0])

            @pl.loop(0, npair)
            def _(p):
                for u in range(2):
                    bb = p * 2 + u
                    cur, nxt = u, 1 - u

                    @pl.when(bb + 1 < npair * 2)
                    def _():
                        @pl.when(bb + 1 >= 2)
                        def _():
                            pltpu.make_async_copy(
                                rows[nxt], agg_sh.at[didx[nxt]],
                                ssem[nxt]).wait()

                        off = (bb + 1) * BB
                        pltpu.sync_copy(csrc_hbm.at[w, pl.ds(off, BB)],
                                        sidx[nxt])
                        pltpu.sync_copy(cdst_hbm.at[w, pl.ds(off, BB)],
                                        didx[nxt])
                        pltpu.async_copy(y2_hbm.at[c].at[sidx[nxt]],
                                         rows[nxt], sems[nxt])

                    pltpu.make_async_copy(y2_hbm.at[c].at[sidx[cur]],
                                          rows[cur], sems[cur]).wait()
                    pltpu.async_copy(rows[cur], agg_sh.at[didx[cur]],
                                     ssem[cur], add=True)

            pltpu.make_async_copy(rows[0], agg_sh.at[didx[0]],
                                  ssem[0]).wait()
            pltpu.make_async_copy(rows[1], agg_sh.at[didx[1]],
                                  ssem[1]).wait()

    plsc.subcore_barrier()
    rows_per = NP // NS
    pltpu.sync_copy(agg_sh.at[pl.ds(s * rows_per, rows_per)],
                    agg_hbm.at[c].at[pl.ds(s * rows_per, rows_per)])


_RB = 1280
_GRID = NP // _RB


def _sum_partials(degp_blk):
    ones = jnp.ones((NW, 1), jnp.float32)
    return lax.dot_general(degp_blk, ones, (((0,), (0,)), ((), ())),
                           preferred_element_type=jnp.float32,
                           precision=lax.Precision.HIGHEST)


def _k2_body(degp_ref, x_ref, y_ref):
    dinv = lax.rsqrt(jnp.maximum(_sum_partials(degp_ref[...]) + 1.0, 1.0))
    y_ref[...] = dinv * x_ref[...]


def _k4a_body(x_ref, aggp_ref, degp_ref, w1_ref, b1_ref, p_ref, h1_ref,
              sc_ref):
    i = pl.program_id(0)
    dinv = lax.rsqrt(jnp.maximum(_sum_partials(degp_ref[...]) + 1.0, 1.0))
    u = dinv * (aggp_ref[0] + aggp_ref[1]) + (dinv * dinv) * x_ref[...]
    h1 = jnp.maximum(
        lax.dot_general(u, w1_ref[...], (((1,), (0,)), ((), ())),
                        preferred_element_type=jnp.float32,
                        precision=lax.Precision.HIGHEST)
        + b1_ref[...][None, :], 0.0)
    h1_ref[...] = h1
    p = p_ref[...]
    pn = jnp.sqrt(jnp.sum(p * p))
    sc = lax.dot_general(h1, p, (((1,), (0,)), ((), ())),
                         preferred_element_type=jnp.float32,
                         precision=lax.Precision.HIGHEST) / pn
    ridx = i * _RB + lax.broadcasted_iota(jnp.int32, (_RB, 1), 0)
    sc_ref[...] = jnp.where(ridx < N, sc, -jnp.inf)


def _k4b_body(sc_ref, kept_ref, gs_ref):
    s = sc_ref[...]
    bits = lax.bitcast_convert_type(s, jnp.int32)
    m = bits ^ jnp.where(bits < 0, jnp.int32(0x7FFFFFFF), jnp.int32(0))

    def bs(it, T):
        cand = T + (jnp.int32(1) << (jnp.int32(30) - it))
        c = jnp.sum((m >= cand).astype(jnp.int32))
        return jnp.where(c >= TOPK, cand, T)

    T = lax.fori_loop(0, 31, bs, jnp.int32(-2**31))
    cnt_gt = jnp.sum((m > T).astype(jnp.int32))
    need = (TOPK - cnt_gt).astype(jnp.float32)
    eq = (m == T)
    eqf = eq.astype(jnp.float32)
    r128 = lax.broadcasted_iota(jnp.int32, (128, 128), 0)
    c128 = lax.broadcasted_iota(jnp.int32, (128, 128), 1)
    u128 = (r128 < c128).astype(jnp.float32)
    p1 = lax.dot_general(eqf, u128, (((1,), (0,)), ((), ())),
                         preferred_element_type=jnp.float32,
                         precision=lax.Precision.HIGHEST)
    rowtot = jnp.sum(eqf, axis=1, keepdims=True)
    r80 = lax.broadcasted_iota(jnp.int32, (80, 80), 0)
    c80 = lax.broadcasted_iota(jnp.int32, (80, 80), 1)
    u80 = (r80 < c80).astype(jnp.float32)
    carry = lax.dot_general(u80, rowtot, (((1,), (0,)), ((), ())),
                            preferred_element_type=jnp.float32,
                            precision=lax.Precision.HIGHEST)
    prefix = p1 + carry
    kept = (m > T) | (eq & (prefix < need))
    keptf = kept.astype(jnp.float32)
    kept_ref[...] = keptf
    gs_ref[...] = keptf * jnp.tanh(s)


def _k4c_body(h1_ref, gs_ref, w3_ref, z_ref):
    xp = gs_ref[...] * h1_ref[...]
    z_ref[...] = lax.dot_general(xp, w3_ref[...], (((1,), (0,)), ((), ())),
                                 preferred_element_type=jnp.float32,
                                 precision=lax.Precision.HIGHEST)


def _dinv2_of(degp_blk, keptf):
    deg2 = _sum_partials(degp_blk) + keptf
    return keptf * lax.rsqrt(jnp.maximum(deg2, 1.0))


def _k6_body(z_ref, degp_ref, kept_ref, y2_ref):
    dinv2 = _dinv2_of(degp_ref[...], kept_ref[...])
    z = z_ref[...]
    y2_ref[0] = dinv2 * z[:, :D]
    y2_ref[1] = dinv2 * z[:, D:]


def _k8_body(agg2_ref, z_ref, degp_ref, kept_ref, batch_ref, b3_ref,
             w2_ref, b2_ref, out_ref, sums, cnt):
    i = pl.program_id(0)

    @pl.when(i == 0)
    def _():
        sums[...] = jnp.zeros_like(sums)
        cnt[...] = jnp.zeros_like(cnt)

    keptf = kept_ref[...]
    dinv2 = _dinv2_of(degp_ref[...], keptf)
    agg = jnp.concatenate([agg2_ref[0], agg2_ref[1]], axis=1)
    h3 = jnp.maximum(dinv2 * agg + (dinv2 * dinv2) * z_ref[...]
                     + b3_ref[...][None, :], 0.0)
    oh = (batch_ref[...] ==
          lax.broadcasted_iota(jnp.int32, (1, NG), 1)).astype(jnp.float32)
    wh3 = keptf * h3
    sums[...] += lax.dot_general(oh, wh3, (((0,), (0,)), ((), ())),
                                 preferred_element_type=jnp.float32,
                                 precision=lax.Precision.HIGHEST)
    cnt[...] += lax.dot_general(oh, keptf, (((0,), (0,)), ((), ())),
                                preferred_element_type=jnp.float32,
                                precision=lax.Precision.HIGHEST)

    @pl.when(i == _GRID - 1)
    def _():
        gm = sums[...] / jnp.maximum(cnt[...], 1.0)
        logit = lax.dot_general(gm, w2_ref[...], (((1,), (0,)), ((), ())),
                                preferred_element_type=jnp.float32,
                                precision=lax.Precision.HIGHEST)
        out_ref[...] = jax.nn.sigmoid(logit + b2_ref[...])


def _rows(block_cols):
    return pl.BlockSpec((_RB, block_cols), lambda i: (i, 0))


def _pair(block_cols):
    return pl.BlockSpec((2, _RB, block_cols), lambda i: (0, i, 0))


def _deg_spec():
    return pl.BlockSpec((NW, _RB), lambda i: (0, i))


def _full(shape):
    return pl.BlockSpec(shape, lambda i: tuple(0 for _ in shape))


def kernel(x, edge_list, dummy, batch, W1, b1, p_vec, W3, b3, W2, b2):
    f32 = jnp.float32
    src = edge_list[:, 0].astype(jnp.int32)
    dst = edge_list[:, 1].astype(jnp.int32)
    pad_e = CHUNK - E // NW
    srcp = jnp.concatenate(
        [src.reshape(NW, E // NW),
         jnp.zeros((NW, pad_e), jnp.int32)], axis=1).reshape(-1)
    dstp = jnp.concatenate(
        [dst.reshape(NW, E // NW),
         jnp.full((NW, pad_e), GR, jnp.int32)], axis=1).reshape(-1)
    xp = jnp.pad(x, ((0, NP - N), (0, 0)))
    batchp = jnp.pad(batch.astype(jnp.int32), (0, NP - N)).reshape(NP, 1)
    zsl = jnp.zeros((16, D), f32)

    deg1p = _k1_deg(dstp)

    y = pl.pallas_call(
        _k2_body,
        grid=(_GRID,),
        in_specs=[_deg_spec(), _rows(D)],
        out_specs=_rows(D),
        out_shape=jax.ShapeDtypeStruct((NP, D), f32),
    )(deg1p, xp)

    agg1p = _k3_agg1(y, srcp, dstp, zsl)

    h1, score = pl.pallas_call(
        _k4a_body,
        grid=(_GRID,),
        in_specs=[_rows(D), _pair(D), _deg_spec(), _full((D, HID)),
                  _full((HID,)), _full((HID, 1))],
        out_specs=[_rows(HID), _rows(1)],
        out_shape=[jax.ShapeDtypeStruct((NP, HID), f32),
                   jax.ShapeDtypeStruct((NP, 1), f32)],
    )(xp, agg1p, deg1p, W1, b1, p_vec.reshape(HID, 1))

    keptm, gsm = pl.pallas_call(
        _k4b_body,
        in_specs=[pl.BlockSpec((80, 128), lambda: (0, 0))],
        out_specs=[pl.BlockSpec((80, 128), lambda: (0, 0))] * 2,
        out_shape=[jax.ShapeDtypeStruct((80, 128), f32)] * 2,
    )(score.reshape(80, 128))
    keptc = keptm.reshape(NP, 1)
    kept_i = keptm.reshape(NP).astype(jnp.int32)

    z = pl.pallas_call(
        _k4c_body,
        grid=(_GRID,),
        in_specs=[_rows(HID), _rows(1), _full((HID, HID))],
        out_specs=_rows(HID),
        out_shape=jax.ShapeDtypeStruct((NP, HID), f32),
    )(h1, gsm.reshape(NP, 1), W3)

    csrc, cdst, cnts, deg2p = _k5_compact(srcp, dstp, kept_i)

    y2 = pl.pallas_call(
        _k6_body,
        grid=(_GRID,),
        in_specs=[_rows(HID), _deg_spec(), _rows(1)],
        out_specs=_pair(D),
        out_shape=jax.ShapeDtypeStruct((NC, NP, D), f32),
    )(z, deg2p, keptc)

    agg2 = _k7_agg2(y2, csrc, cdst, cnts, zsl)

    out = pl.pallas_call(
        _k8_body,
        grid=(_GRID,),
        in_specs=[_pair(D), _rows(HID), _deg_spec(), _rows(1),
                  pl.BlockSpec((_RB, 1), lambda i: (i, 0)),
                  _full((HID,)), _full((HID, 1)), _full((1, 1))],
        out_specs=_full((NG, 1)),
        out_shape=jax.ShapeDtypeStruct((NG, 1), f32),
        scratch_shapes=[pltpu.VMEM((NG, HID), f32),
                        pltpu.VMEM((NG, 1), f32)],
    )(agg2, z, deg2p, keptc, batchp, b3, W2, b2.reshape(1, 1))
    return out.reshape(-1)
